# baseline (device time: 22041 ns/iter reference)
import jax
import jax.numpy as jnp
from jax import lax
from jax.experimental import pallas as pl
from jax.experimental.pallas import tpu as pltpu

N_DEV = 16


def _a2a(x):
    m_full, k_per = x.shape
    m_per = m_full // N_DEV

    def body(x_ref, out_ref, sendbuf_ref, send_sems, recv_sems, ready_sems):
        me = lax.axis_index("i")

        barrier_sem = pltpu.get_barrier_semaphore()
        pl.semaphore_signal(barrier_sem, inc=1)
        pl.semaphore_wait(barrier_sem, 1)

        for d in range(1, N_DEV):
            pl.semaphore_signal(
                ready_sems.at[d], inc=1,
                device_id=(lax.rem(me - d + N_DEV, N_DEV),),
                device_id_type=pl.DeviceIdType.MESH,
            )

        sendbuf_ref[...] = x_ref[...].reshape(N_DEV, m_per, k_per).astype(
            jnp.bfloat16
        )

        copies = []
        for d in range(1, N_DEV):
            dst = lax.rem(me + d, N_DEV)
            pl.semaphore_wait(ready_sems.at[d], 1)
            rdma = pltpu.make_async_remote_copy(
                src_ref=sendbuf_ref.at[dst],
                dst_ref=out_ref.at[me],
                send_sem=send_sems.at[d],
                recv_sem=recv_sems.at[d],
                device_id=(dst,),
                device_id_type=pl.DeviceIdType.MESH,
            )
            rdma.start()
            copies.append(rdma)

        out_ref[me] = sendbuf_ref[me]

        for rdma in copies:
            rdma.wait_recv()
        for rdma in copies:
            rdma.wait_send()

    return pl.pallas_call(
        body,
        out_shape=jax.ShapeDtypeStruct((N_DEV, m_per, k_per), jnp.bfloat16),
        in_specs=[pl.BlockSpec(memory_space=pltpu.VMEM)],
        out_specs=pl.BlockSpec(memory_space=pltpu.VMEM),
        scratch_shapes=[
            pltpu.VMEM((N_DEV, m_per, k_per), jnp.bfloat16),
            pltpu.SemaphoreType.DMA((N_DEV,)),
            pltpu.SemaphoreType.DMA((N_DEV,)),
            pltpu.SemaphoreType.REGULAR((N_DEV,)),
        ],
        compiler_params=pltpu.CompilerParams(collective_id=0),
    )(x)


def _gemm(blocks, w_mat):
    _, m_per, k_per = blocks.shape
    _, n = w_mat.shape

    def body(b_ref, w_ref, out_ref):
        acc = None
        for j in range(N_DEV):
            t = jnp.dot(
                b_ref[j].astype(jnp.float32),
                w_ref[pl.ds(j * k_per, k_per), :],
                preferred_element_type=jnp.float32,
            )
            acc = t if acc is None else acc + t
        out_ref[...] = acc

    return pl.pallas_call(
        body,
        out_shape=jax.ShapeDtypeStruct((m_per, n), jnp.float32),
        in_specs=[
            pl.BlockSpec(memory_space=pltpu.VMEM),
            pl.BlockSpec(memory_space=pltpu.VMEM),
        ],
        out_specs=pl.BlockSpec(memory_space=pltpu.VMEM),
    )(blocks, w_mat)


def kernel(x, w_mat):
    return _gemm(_a2a(x), w_mat)


# device time: 20576 ns/iter; 1.0712x vs baseline; 1.0712x over previous
import jax
import jax.numpy as jnp
from jax import lax
from jax.experimental import pallas as pl
from jax.experimental.pallas import tpu as pltpu

N_DEV = 16


def kernel(x, w_mat):
    m_full, k_per = x.shape
    k_full, n = w_mat.shape
    m_per = m_full // N_DEV

    def body(x_ref, w_ref, out_ref, sendbuf_ref, recvbuf_ref, w16_ref,
             send_sems, recv_sems, ready_sems):
        me = lax.axis_index("i")

        barrier_sem = pltpu.get_barrier_semaphore()
        pl.semaphore_signal(barrier_sem, inc=1)
        pl.semaphore_wait(barrier_sem, 1)

        for d in range(1, N_DEV):
            pl.semaphore_signal(
                ready_sems.at[d], inc=1,
                device_id=(lax.rem(me - d + N_DEV, N_DEV),),
                device_id_type=pl.DeviceIdType.MESH,
            )

        sendbuf_ref[...] = x_ref[...].reshape(N_DEV, m_per, k_per).astype(
            jnp.bfloat16
        )

        order = []
        for i in range(1, N_DEV // 2):
            order += [i, N_DEV - i]
        order.append(N_DEV // 2)

        copies = {}
        for d in order:
            dst = lax.rem(me + d, N_DEV)
            pl.semaphore_wait(ready_sems.at[d], 1)
            rdma = pltpu.make_async_remote_copy(
                src_ref=sendbuf_ref.at[dst],
                dst_ref=recvbuf_ref.at[me],
                send_sem=send_sems.at[d],
                recv_sem=recv_sems.at[d],
                device_id=(dst,),
                device_id_type=pl.DeviceIdType.MESH,
            )
            rdma.start()
            copies[d] = rdma

        w16_ref[...] = w_ref[...].astype(jnp.bfloat16)

        acc = jnp.dot(
            sendbuf_ref[me],
            w16_ref[pl.ds(me * k_per, k_per), :],
            preferred_element_type=jnp.float32,
        )

        for d in order:
            copies[d].wait_recv()
            j = lax.rem(me - d + N_DEV, N_DEV)
            acc = acc + jnp.dot(
                recvbuf_ref[j],
                w16_ref[pl.ds(j * k_per, k_per), :],
                preferred_element_type=jnp.float32,
            )
        out_ref[...] = acc

        for rdma in copies.values():
            rdma.wait_send()

    return pl.pallas_call(
        body,
        out_shape=jax.ShapeDtypeStruct((m_per, n), jnp.float32),
        in_specs=[
            pl.BlockSpec(memory_space=pltpu.VMEM),
            pl.BlockSpec(memory_space=pltpu.VMEM),
        ],
        out_specs=pl.BlockSpec(memory_space=pltpu.VMEM),
        scratch_shapes=[
            pltpu.VMEM((N_DEV, m_per, k_per), jnp.bfloat16),
            pltpu.VMEM((N_DEV, m_per, k_per), jnp.bfloat16),
            pltpu.VMEM((k_full, n), jnp.bfloat16),
            pltpu.SemaphoreType.DMA((N_DEV,)),
            pltpu.SemaphoreType.DMA((N_DEV,)),
            pltpu.SemaphoreType.REGULAR((N_DEV,)),
        ],
        compiler_params=pltpu.CompilerParams(collective_id=0),
    )(x, w_mat)
